# baseline (speedup 1.0000x reference)
import functools

import jax
import jax.numpy as jnp
from jax import lax
from jax.experimental import pallas as pl
from jax.experimental.pallas import tpu as pltpu
from jax.experimental.pallas import tpu_sc as plsc

_N_EMBED = 512
_E_DIM = 64
_BETA = 0.25

_BLK = 2048


def _tc_body(z_ref, e_ref, e2c_ref, rowsf_ref, idx_ref, loss_ref):
    i = pl.program_id(0)
    z = z_ref[...]
    e = e_ref[...]
    simT = lax.dot_general(e, z, (((1,), (1,)), ((), ())),
                           preferred_element_type=jnp.float32,
                           precision=lax.Precision.DEFAULT)
    colmax = jnp.max(simT, axis=0, keepdims=True)
    mask = simT == colmax
    rowsf = rowsf_ref[...]
    idx_f = jnp.min(jnp.where(mask, rowsf, jnp.float32(_N_EMBED)),
                    axis=0, keepdims=True)
    idx_ref[...] = idx_f.astype(jnp.int32)[None]

    e2c = e2c_ref[...]
    e2_sel = jnp.min(jnp.where(mask, e2c, jnp.inf), axis=0)
    part = (jnp.sum(e2_sel) - 2.0 * jnp.sum(colmax) + jnp.sum(z * z))
    loss_ref[0, 0] = jnp.where(i == 0, part, loss_ref[0, 0] + part)


def _make_sc_gather():
    info = plsc.get_sparse_core_info()
    nw = info.num_cores * info.num_subcores
    rows_per_w = 65536 // nw
    n_idx_rows = rows_per_w // 128
    chunk = 4
    chunk_rows = chunk * 128
    n_chunks = n_idx_rows // chunk

    mesh = plsc.VectorSubcoreMesh(core_axis_name="c", subcore_axis_name="s")

    @functools.partial(
        pl.kernel, mesh=mesh,
        compiler_params=pltpu.CompilerParams(use_tc_tiling_on_sc=False),
        out_type=jax.ShapeDtypeStruct((65536, _E_DIM), jnp.float32),
        scratch_types=[
            pltpu.VMEM((n_idx_rows, 128), jnp.int32),
            pltpu.VMEM((chunk_rows, _E_DIM), jnp.float32),
            pltpu.VMEM((chunk_rows, _E_DIM), jnp.float32),
            pltpu.SemaphoreType.DMA,
            pltpu.SemaphoreType.DMA,
            pltpu.SemaphoreType.DMA,
        ],
    )
    def sc_gather(table_hbm, idx_hbm, out_hbm, idx_v, rows_a, rows_b,
                  gsem, ssem_a, ssem_b):
        wid = lax.axis_index("s") * info.num_cores + lax.axis_index("c")
        pltpu.sync_copy(idx_hbm.at[pl.ds(wid * n_idx_rows, n_idx_rows)], idx_v)
        bufs = [rows_a, rows_b]
        ssems = [ssem_a, ssem_b]
        scatters = [None, None]
        for c in range(n_chunks):
            b = c & 1
            if scatters[b] is not None:
                scatters[b].wait()
            cps = [
                pltpu.async_copy(
                    table_hbm.at[idx_v.at[c * chunk + j]],
                    bufs[b].at[pl.ds(j * 128, 128)],
                    gsem,
                )
                for j in range(chunk)
            ]
            for cp in cps:
                cp.wait()
            scatters[b] = pltpu.async_copy(
                bufs[b],
                out_hbm.at[pl.ds(wid * rows_per_w + c * chunk_rows,
                                 chunk_rows)],
                ssems[b],
            )
        for sc in scatters:
            sc.wait()

    return sc_gather


_sc_gather = None


def kernel(z, embedding_weight):
    global _sc_gather
    if _sc_gather is None:
        _sc_gather = _make_sc_gather()

    z_flat = z.reshape(-1, _E_DIM)
    e2c = jnp.sum(embedding_weight * embedding_weight, axis=1)[:, None]
    rowsf = jnp.arange(_N_EMBED, dtype=jnp.float32)[:, None]

    n_rows = z_flat.shape[0]
    n_blk = n_rows // _BLK
    idx2, loss_sum = pl.pallas_call(
        _tc_body,
        grid=(n_blk,),
        in_specs=[
            pl.BlockSpec((_BLK, _E_DIM), lambda i: (i, 0)),
            pl.BlockSpec((_N_EMBED, _E_DIM), lambda i: (0, 0)),
            pl.BlockSpec((_N_EMBED, 1), lambda i: (0, 0)),
            pl.BlockSpec((_N_EMBED, 1), lambda i: (0, 0)),
        ],
        out_specs=[
            pl.BlockSpec((1, 1, _BLK), lambda i: (i, 0, 0)),
            pl.BlockSpec(memory_space=pltpu.SMEM, index_map=lambda i: (0, 0)),
        ],
        out_shape=[
            jax.ShapeDtypeStruct((n_blk, 1, _BLK), jnp.int32),
            jax.ShapeDtypeStruct((1, 1), jnp.float32),
        ],
    )(z_flat, embedding_weight, e2c, rowsf)

    idx = idx2.reshape(-1)
    idx2d = idx.reshape(512, 128)
    z_q_flat = _sc_gather(embedding_weight, idx2d)
    z_q = z_q_flat.reshape(z.shape)

    mse = loss_sum[0, 0] / jnp.float32(z.size)
    vq_loss = mse
    commitment_loss = _BETA * mse
    return (z_q, vq_loss, commitment_loss, idx)


# speedup vs baseline: 1.0276x; 1.0276x over previous
import functools

import jax
import jax.numpy as jnp
from jax import lax
from jax.experimental import pallas as pl
from jax.experimental.pallas import tpu as pltpu
from jax.experimental.pallas import tpu_sc as plsc

_N_EMBED = 512
_E_DIM = 64
_BETA = 0.25

_BLK = 2048


def _tc_body(z_ref, e_ref, e2c_ref, rowsf_ref, idx_ref, loss_ref):
    i = pl.program_id(0)
    z = z_ref[...]
    e = e_ref[...]
    simT = lax.dot_general(e, z, (((1,), (1,)), ((), ())),
                           preferred_element_type=jnp.float32,
                           precision=lax.Precision.DEFAULT)
    colmax = jnp.max(simT, axis=0, keepdims=True)
    mask = simT == colmax
    rowsf = rowsf_ref[...]
    idx_f = jnp.min(jnp.where(mask, rowsf, jnp.float32(_N_EMBED)),
                    axis=0, keepdims=True)
    idx_ref[...] = idx_f.astype(jnp.int32).reshape(1, _BLK // 128, 128)

    e2c = e2c_ref[...]
    e2_sel = jnp.min(jnp.where(mask, e2c, jnp.inf), axis=0)
    part = (jnp.sum(e2_sel) - 2.0 * jnp.sum(colmax) + jnp.sum(z * z))
    loss_ref[0, 0] = jnp.where(i == 0, part, loss_ref[0, 0] + part)


def _make_sc_gather():
    info = plsc.get_sparse_core_info()
    nw = info.num_cores * info.num_subcores
    rows_per_w = 65536 // nw
    n_idx_rows = rows_per_w // 128
    chunk = 4
    chunk_rows = chunk * 128
    n_chunks = n_idx_rows // chunk

    mesh = plsc.VectorSubcoreMesh(core_axis_name="c", subcore_axis_name="s")

    @functools.partial(
        pl.kernel, mesh=mesh,
        compiler_params=pltpu.CompilerParams(use_tc_tiling_on_sc=False),
        out_type=jax.ShapeDtypeStruct((65536, _E_DIM), jnp.float32),
        scratch_types=[
            pltpu.VMEM((n_idx_rows, 128), jnp.int32),
            pltpu.VMEM((chunk_rows, _E_DIM), jnp.float32),
            pltpu.VMEM((chunk_rows, _E_DIM), jnp.float32),
            pltpu.VMEM((chunk_rows, _E_DIM), jnp.float32),
            pltpu.SemaphoreType.DMA,
            pltpu.SemaphoreType.DMA,
            pltpu.SemaphoreType.DMA,
            pltpu.SemaphoreType.DMA,
            pltpu.SemaphoreType.DMA,
            pltpu.SemaphoreType.DMA,
        ],
    )
    def sc_gather(table_hbm, idx_hbm, out_hbm, idx_v, rows_a, rows_b, rows_c,
                  gsem_a, gsem_b, gsem_c, ssem_a, ssem_b, ssem_c):
        wid = lax.axis_index("s") * info.num_cores + lax.axis_index("c")
        pltpu.sync_copy(idx_hbm.at[pl.ds(wid * n_idx_rows, n_idx_rows)], idx_v)
        nbuf = 3
        bufs = [rows_a, rows_b, rows_c]
        gsems = [gsem_a, gsem_b, gsem_c]
        ssems = [ssem_a, ssem_b, ssem_c]

        def fire_gathers(c):
            b = c % nbuf
            return [
                pltpu.async_copy(
                    table_hbm.at[idx_v.at[c * chunk + j]],
                    bufs[b].at[pl.ds(j * 128, 128)],
                    gsems[b],
                )
                for j in range(chunk)
            ]

        gathers = {}
        scatters = [None, None, None]
        for c in range(min(nbuf, n_chunks)):
            gathers[c] = fire_gathers(c)
        for c in range(n_chunks):
            b = c % nbuf
            for cp in gathers.pop(c):
                cp.wait()
            scatters[b] = pltpu.async_copy(
                bufs[b],
                out_hbm.at[pl.ds(wid * rows_per_w + c * chunk_rows,
                                 chunk_rows)],
                ssems[b],
            )
            nxt = c + nbuf
            if nxt < n_chunks:
                scatters[b].wait()
                scatters[b] = None
                gathers[nxt] = fire_gathers(nxt)
        for sc in scatters:
            if sc is not None:
                sc.wait()

    return sc_gather


_sc_gather = None


def kernel(z, embedding_weight):
    global _sc_gather
    if _sc_gather is None:
        _sc_gather = _make_sc_gather()

    z_flat = z.reshape(-1, _E_DIM)
    e2c = jnp.sum(embedding_weight * embedding_weight, axis=1)[:, None]
    rowsf = jnp.arange(_N_EMBED, dtype=jnp.float32)[:, None]

    n_rows = z_flat.shape[0]
    n_blk = n_rows // _BLK
    idx2, loss_sum = pl.pallas_call(
        _tc_body,
        grid=(n_blk,),
        in_specs=[
            pl.BlockSpec((_BLK, _E_DIM), lambda i: (i, 0)),
            pl.BlockSpec((_N_EMBED, _E_DIM), lambda i: (0, 0)),
            pl.BlockSpec((_N_EMBED, 1), lambda i: (0, 0)),
            pl.BlockSpec((_N_EMBED, 1), lambda i: (0, 0)),
        ],
        out_specs=[
            pl.BlockSpec((1, _BLK // 128, 128), lambda i: (i, 0, 0)),
            pl.BlockSpec(memory_space=pltpu.SMEM, index_map=lambda i: (0, 0)),
        ],
        out_shape=[
            jax.ShapeDtypeStruct((n_blk, _BLK // 128, 128), jnp.int32),
            jax.ShapeDtypeStruct((1, 1), jnp.float32),
        ],
    )(z_flat, embedding_weight, e2c, rowsf)

    idx = idx2.reshape(-1)
    idx2d = idx.reshape(512, 128)
    z_q_flat = _sc_gather(embedding_weight, idx2d)
    z_q = z_q_flat.reshape(z.shape)

    mse = loss_sum[0, 0] / jnp.float32(z.size)
    vq_loss = mse
    commitment_loss = _BETA * mse
    return (z_q, vq_loss, commitment_loss, idx)
